# baseline (speedup 1.0000x reference)
import functools

import jax
import jax.numpy as jnp
from jax import lax
from jax.experimental import pallas as pl
from jax.experimental.pallas import tpu as pltpu
from jax.experimental.pallas import tpu_sc as plsc

VOCAB = 16384
HIDDEN = 1024
SEQ = 128
BATCH = 4096

NC, NS, L = 2, 16, 16
NW = NC * NS
ROWS = BATCH * SEQ
ROWS_PER_W = ROWS // NW
SEQS_PER_W = BATCH // NW
CHUNK = 16
JC = SEQ // CHUNK
NBUF = 4
CHUNKS_PER_W = ROWS_PER_W // CHUNK
UNROLL = 2


def _sc_kernel(ids_hbm, table_hbm, pos_hbm, out_hbm,
               ixq, pos_v, b0, b1, b2, b3,
               iq, g0, g1, g2, g3, s0, s1, s2, s3):
    bufs = [b0, b1, b2, b3]
    gsem = [g0, g1, g2, g3]
    ssem = [s0, s1, s2, s3]

    cid = lax.axis_index("c")
    sid = lax.axis_index("s")
    wid = sid * NC + cid
    wbase = wid * ROWS_PER_W
    idbase = wid * CHUNKS_PER_W

    def gather(s, b):
        pltpu.async_copy(table_hbm.at[ixq.at[s]], bufs[b], gsem[b])

    def add_pos(b):
        rows_v = bufs[b]

        def col_body(k, _):
            xs = []
            for u in range(UNROLL):
                cc = k * UNROLL + u
                t = cc // (HIDDEN // 8 // L)
                sl = pl.ds(lax.rem(cc, HIDDEN // 8 // L) * L, L)
                for r in range(CHUNK):
                    xs.append((r, t, sl, pos_v[r, t, sl]))
            for r, t, sl, x in xs:
                plsc.addupdate(rows_v.at[r, t, sl], x)
            return 0

        lax.fori_loop(0, HIDDEN // L // UNROLL, col_body, 0)

    def finish(jc, s, b):
        pltpu.make_async_copy(table_hbm.at[pl.ds(0, CHUNK)], bufs[b],
                              gsem[b]).wait()
        add_pos(b)
        base = wbase + s * SEQ + jc * CHUNK
        for t in range(8):
            pltpu.async_copy(
                bufs[b].at[:, t],
                out_hbm.at[pl.ds(base, CHUNK), pl.ds(t * 128, 128)], ssem[b])

    def wait_store(b):
        pltpu.make_async_copy(table_hbm.at[pl.ds(0, CHUNK)], bufs[b],
                              ssem[b]).wait()

    def jc_body(jc, _):
        pltpu.sync_copy(ids_hbm.at[pl.ds(idbase + jc * SEQS_PER_W, SEQS_PER_W)],
                        ixq)
        pltpu.sync_copy(pos_hbm.at[pl.ds(jc * CHUNK, CHUNK)], pos_v)

        gather(0, 0)
        gather(1, 1)
        gather(2, 2)
        gather(3, 3)
        finish(jc, 0, 0)
        finish(jc, 1, 1)

        def round_body(r, _, jc=jc):
            for b in range(NBUF):
                s = r * NBUF + b
                wait_store(b)
                gather(s, b)
                finish(jc, s - 2, (b - 2) % NBUF)
            return 0

        lax.fori_loop(1, SEQS_PER_W // NBUF, round_body, 0)

        finish(jc, SEQS_PER_W - 2, (SEQS_PER_W - 2) % NBUF)
        finish(jc, SEQS_PER_W - 1, (SEQS_PER_W - 1) % NBUF)
        for b in range(NBUF):
            wait_store(b)
        return 0

    lax.fori_loop(0, JC, jc_body, 0)


@jax.jit
def _run(ids2, table, pos):
    mesh = plsc.VectorSubcoreMesh(core_axis_name="c", subcore_axis_name="s")
    f = functools.partial(
        pl.kernel,
        mesh=mesh,
        out_type=jax.ShapeDtypeStruct((ROWS, HIDDEN), jnp.float32),
        scratch_types=[
            pltpu.VMEM((SEQS_PER_W, CHUNK), jnp.int32),
            pltpu.VMEM((CHUNK, 8, HIDDEN // 8), jnp.float32),
            pltpu.VMEM((CHUNK, 8, HIDDEN // 8), jnp.float32),
            pltpu.VMEM((CHUNK, 8, HIDDEN // 8), jnp.float32),
            pltpu.VMEM((CHUNK, 8, HIDDEN // 8), jnp.float32),
            pltpu.VMEM((CHUNK, 8, HIDDEN // 8), jnp.float32),
        ] + [pltpu.SemaphoreType.DMA] * 9,
    )(_sc_kernel)
    return f(ids2, table, pos)


def kernel(input_ids, text_embeddings, text_pos_embeddings):
    ids4 = input_ids.astype(jnp.int32).reshape(NW, SEQS_PER_W, JC, CHUNK)
    ids2 = ids4.transpose(0, 2, 1, 3).reshape(CHUNKS_PER_W * NW, CHUNK)
    table3 = text_embeddings.reshape(VOCAB, 8, HIDDEN // 8)
    pos3 = text_pos_embeddings[:SEQ].reshape(SEQ, 8, HIDDEN // 8)
    out = _run(ids2, table3, pos3)
    return out.reshape(BATCH, SEQ, HIDDEN)


# speedup vs baseline: 1.1626x; 1.1626x over previous
import functools

import jax
import jax.numpy as jnp
from jax import lax
from jax.experimental import pallas as pl
from jax.experimental.pallas import tpu as pltpu
from jax.experimental.pallas import tpu_sc as plsc

VOCAB = 16384
HIDDEN = 1024
SEQ = 128
BATCH = 4096

NC, NS, L = 2, 16, 16
NW = NC * NS
ROWS = BATCH * SEQ
ROWS_PER_W = ROWS // NW
SEQS_PER_W = BATCH // NW
CHUNK = 16
JC = SEQ // CHUNK
NBUF = 4
CHUNKS_PER_W = ROWS_PER_W // CHUNK
UNROLL = 2


def _sc_kernel(ids_hbm, table_hbm, pos_hbm, out_hbm,
               ixq, pos_v, b0, b1, b2, b3,
               iq, g0, g1, g2, g3, s0, s1, s2, s3):
    bufs = [b0, b1, b2, b3]
    gsem = [g0, g1, g2, g3]
    ssem = [s0, s1, s2, s3]

    cid = lax.axis_index("c")
    sid = lax.axis_index("s")
    wid = sid * NC + cid
    wbase = wid * ROWS_PER_W
    idbase = wid * CHUNKS_PER_W

    def gather(s, b):
        pltpu.async_copy(table_hbm.at[ixq.at[s]], bufs[b], gsem[b])

    def add_pos(b):
        rows_v = bufs[b]

        def col_body(c, _):
            xs = []
            for u in range(UNROLL):
                sl = pl.ds((c * UNROLL + u) * L, L)
                for r in range(CHUNK):
                    xs.append((r, sl, pos_v[r, sl]))
            for r, sl, x in xs:
                plsc.addupdate(rows_v.at[r, sl], x)
            return 0

        lax.fori_loop(0, HIDDEN // L // UNROLL, col_body, 0)

    def finish(jc, s, b):
        pltpu.make_async_copy(table_hbm.at[pl.ds(0, CHUNK)], bufs[b],
                              gsem[b]).wait()
        add_pos(b)
        base = wbase + s * SEQ + jc * CHUNK
        pltpu.async_copy(bufs[b], out_hbm.at[pl.ds(base, CHUNK)], ssem[b])

    def wait_store(b):
        pltpu.make_async_copy(bufs[b], out_hbm.at[pl.ds(0, CHUNK)], ssem[b]).wait()

    def jc_body(jc, _):
        pltpu.sync_copy(ids_hbm.at[pl.ds(idbase + jc * SEQS_PER_W, SEQS_PER_W)],
                        ixq)
        pltpu.sync_copy(pos_hbm.at[pl.ds(jc * CHUNK, CHUNK)], pos_v)

        gather(0, 0)
        gather(1, 1)
        gather(2, 2)
        gather(3, 3)
        finish(jc, 0, 0)
        finish(jc, 1, 1)

        def round_body(r, _, jc=jc):
            for b in range(NBUF):
                s = r * NBUF + b
                wait_store(b)
                gather(s, b)
                finish(jc, s - 2, (b - 2) % NBUF)
            return 0

        lax.fori_loop(1, SEQS_PER_W // NBUF, round_body, 0)

        finish(jc, SEQS_PER_W - 2, (SEQS_PER_W - 2) % NBUF)
        finish(jc, SEQS_PER_W - 1, (SEQS_PER_W - 1) % NBUF)
        for b in range(NBUF):
            wait_store(b)
        return 0

    lax.fori_loop(0, JC, jc_body, 0)


@jax.jit
def _run(ids2, table, pos):
    mesh = plsc.VectorSubcoreMesh(core_axis_name="c", subcore_axis_name="s")
    f = functools.partial(
        pl.kernel,
        mesh=mesh,
        out_type=jax.ShapeDtypeStruct((ROWS, HIDDEN), jnp.float32),
        scratch_types=[
            pltpu.VMEM((SEQS_PER_W, CHUNK), jnp.int32),
            pltpu.VMEM((CHUNK, HIDDEN), jnp.float32),
            pltpu.VMEM((CHUNK, HIDDEN), jnp.float32),
            pltpu.VMEM((CHUNK, HIDDEN), jnp.float32),
            pltpu.VMEM((CHUNK, HIDDEN), jnp.float32),
            pltpu.VMEM((CHUNK, HIDDEN), jnp.float32),
        ] + [pltpu.SemaphoreType.DMA] * 9,
    )(_sc_kernel)
    return f(ids2, table, pos)


def kernel(input_ids, text_embeddings, text_pos_embeddings):
    ids4 = input_ids.astype(jnp.int32).reshape(NW, SEQS_PER_W, JC, CHUNK)
    ids2 = ids4.transpose(0, 2, 1, 3).reshape(CHUNKS_PER_W * NW, CHUNK)
    out = _run(ids2, text_embeddings, text_pos_embeddings)
    return out.reshape(BATCH, SEQ, HIDDEN)
